# Initial kernel scaffold; baseline (speedup 1.0000x reference)
import functools

import jax
import jax.numpy as jnp
from jax import lax
from jax.experimental import pallas as pl
from jax.experimental.pallas import tpu as pltpu
from jax.experimental.pallas import tpu_sc as plsc

B = 4
S = 8192
S1 = S + 1
D = 1024
HALF = D // 2
NPOS = 101
NC, NS = 2, 16
NW = NC * NS
RPT = S // NW
CH = 16
NCHUNK = RPT // CH
BASER_LEN = 1152
MCH = 512


def _body(x_hbm, pos_hbm, pe_hbm, out_hbm, petab, baser, xbuf, posb, pbuf):
    wid = lax.axis_index("s") * NC + lax.axis_index("c")
    iota = lax.iota(jnp.int32, 16)
    zeros16 = jnp.zeros((16,), jnp.int32)

    pltpu.sync_copy(pe_hbm, petab)

    @pl.when(wid < B)
    def _copy_row0():
        r = wid * S1
        pltpu.sync_copy(x_hbm.at[pl.ds(r, 1), :], xbuf.at[pl.ds(0, 1), :])
        pltpu.sync_copy(xbuf.at[pl.ds(0, 1), :], out_hbm.at[pl.ds(r, 1), :])

    for b in range(B):
        def scan_chunk(c, mv):
            pltpu.sync_copy(pos_hbm.at[pl.ds(b * S + c * MCH, MCH), :], pbuf)

            def gmax(u, mv2):
                vals = plsc.load_gather(pbuf, [iota + u * 16, zeros16])
                return jnp.maximum(mv2, vals)

            return lax.fori_loop(0, MCH // 16, gmax, mv)

        mvec = lax.fori_loop(0, S // MCH, scan_chunk,
                             jnp.full((16,), -1, jnp.int32))
        m = jnp.max(mvec)
        step = m + 1

        def fill(u, _):
            t = iota + u * 16
            d = (D - 1) - t
            cond = (lax.rem(d, step) == 0) & (d >= 0) & (m > 0)
            baser[b, pl.ds(u * 16, 16)] = jnp.where(cond, 1.0, 0.0).astype(
                jnp.float32)
            return 0

        lax.fori_loop(0, BASER_LEN // 16, fill, 0)

        bsplat = jnp.full((16,), b, jnp.int32)

        def do_chunk(k, _):
            posrow = b * S + wid * RPT + k * CH
            xrow = b * S1 + 1 + wid * RPT + k * CH
            pltpu.sync_copy(pos_hbm.at[pl.ds(posrow, CH), :], posb)
            pltpu.sync_copy(x_hbm.at[pl.ds(xrow, CH), :], xbuf)

            def row(i, _):
                isplat = jnp.full((16,), i, jnp.int32)
                p0 = plsc.load_gather(posb, [isplat, zeros16])
                p1 = plsc.load_gather(posb, [isplat, zeros16 + 1])
                p2 = plsc.load_gather(posb, [isplat, zeros16 + 2])
                for v in range(D // 16):
                    ccol = iota + v * 16
                    if v < HALF // 16:
                        pev = plsc.load_gather(petab, [p1, ccol])
                    else:
                        pev = plsc.load_gather(petab, [p2, ccol - HALF])
                    secv = plsc.load_gather(baser, [bsplat, p0 + ccol])
                    xv = xbuf[i, pl.ds(v * 16, 16)]
                    xbuf[i, pl.ds(v * 16, 16)] = xv + pev + secv
                return 0

            lax.fori_loop(0, CH, row, 0)
            pltpu.sync_copy(xbuf, out_hbm.at[pl.ds(xrow, CH), :])
            return 0

        lax.fori_loop(0, NCHUNK, do_chunk, 0)


_sc_call = functools.partial(
    pl.kernel,
    out_type=jax.ShapeDtypeStruct((B * S1, D), jnp.float32),
    mesh=plsc.VectorSubcoreMesh(core_axis_name="c", subcore_axis_name="s"),
    scratch_types=[
        pltpu.VMEM((NPOS, HALF), jnp.float32),
        pltpu.VMEM((B, BASER_LEN), jnp.float32),
        pltpu.VMEM((CH, D), jnp.float32),
        pltpu.VMEM((CH, 3), jnp.int32),
        pltpu.VMEM((MCH, 3), jnp.int32),
    ],
)(_body)


def kernel(x, pos, pos_embed):
    x2 = x.reshape(B * S1, D)
    pos2 = pos.reshape(B * S, 3).astype(jnp.int32)
    out2 = _sc_call(x2, pos2, pos_embed)
    return out2.reshape(B, S1, D)


# speedup vs baseline: 1.2565x; 1.2565x over previous
import functools

import jax
import jax.numpy as jnp
from jax import lax
from jax.experimental import pallas as pl
from jax.experimental.pallas import tpu as pltpu
from jax.experimental.pallas import tpu_sc as plsc

B = 4
S = 8192
S1 = S + 1
D = 1024
HALF = D // 2
NPOS = 101
NC, NS = 2, 16
NW = NC * NS
RPT = S // NW
CH = 16
NCHUNK = RPT // CH
BASER_LEN = 1152
MCH = 512


def _body(x_hbm, pos_hbm, pe_hbm, out_hbm, petab, baser, xbuf, posb, pbuf):
    wid = lax.axis_index("s") * NC + lax.axis_index("c")
    iota = lax.iota(jnp.int32, 16)

    pltpu.sync_copy(pe_hbm, petab)

    @pl.when(wid < B)
    def _copy_row0():
        r = wid * (S1 * D)
        pltpu.sync_copy(x_hbm.at[pl.ds(r, D)], xbuf.at[pl.ds(0, D)])
        pltpu.sync_copy(xbuf.at[pl.ds(0, D)], out_hbm.at[pl.ds(r, D)])

    for b in range(B):
        def scan_chunk(c, mv):
            pltpu.sync_copy(
                pos_hbm.at[pl.ds((b * S + c * MCH) * 3, MCH * 3)], pbuf)

            def gmax(u, mv2):
                vals = plsc.load_gather(pbuf, [(iota + u * 16) * 3])
                return jnp.maximum(mv2, vals)

            return lax.fori_loop(0, MCH // 16, gmax, mv)

        mvec = lax.fori_loop(0, S // MCH, scan_chunk,
                             jnp.full((16,), -1, jnp.int32))
        m = jnp.max(mvec)
        step = m + 1

        def fill(u, _):
            t = iota + u * 16
            d = (D - 1) - t
            cond = (lax.rem(d, step) == 0) & (d >= 0) & (m > 0)
            baser[pl.ds(b * BASER_LEN + u * 16, 16)] = jnp.where(
                cond, 1.0, 0.0).astype(jnp.float32)
            return 0

        lax.fori_loop(0, BASER_LEN // 16, fill, 0)

        def do_chunk(k, _):
            posrow = b * S + wid * RPT + k * CH
            xrow = b * S1 + 1 + wid * RPT + k * CH
            pltpu.sync_copy(pos_hbm.at[pl.ds(posrow * 3, CH * 3)], posb)
            pltpu.sync_copy(x_hbm.at[pl.ds(xrow * D, CH * D)], xbuf)

            def row(i, _):
                i3 = jnp.full((16,), 0, jnp.int32) + i * 3
                p0 = plsc.load_gather(posb, [i3])
                p1 = plsc.load_gather(posb, [i3 + 1])
                p2 = plsc.load_gather(posb, [i3 + 2])
                sbase = p0 + b * BASER_LEN
                for v in range(D // 16):
                    ccol = iota + v * 16
                    if v < HALF // 16:
                        pev = plsc.load_gather(petab, [p1 * HALF + ccol])
                    else:
                        pev = plsc.load_gather(petab,
                                               [p2 * HALF + (ccol - HALF)])
                    secv = plsc.load_gather(baser, [sbase + ccol])
                    xv = xbuf[pl.ds(i * D + v * 16, 16)]
                    xbuf[pl.ds(i * D + v * 16, 16)] = xv + pev + secv
                return 0

            lax.fori_loop(0, CH, row, 0)
            pltpu.sync_copy(xbuf, out_hbm.at[pl.ds(xrow * D, CH * D)])
            return 0

        lax.fori_loop(0, NCHUNK, do_chunk, 0)


_sc_call = functools.partial(
    pl.kernel,
    out_type=jax.ShapeDtypeStruct((B * S1 * D,), jnp.float32),
    mesh=plsc.VectorSubcoreMesh(core_axis_name="c", subcore_axis_name="s"),
    compiler_params=pltpu.CompilerParams(use_tc_tiling_on_sc=False,
                                         needs_layout_passes=False),
    scratch_types=[
        pltpu.VMEM((NPOS * HALF,), jnp.float32),
        pltpu.VMEM((B * BASER_LEN,), jnp.float32),
        pltpu.VMEM((CH * D,), jnp.float32),
        pltpu.VMEM((CH * 3,), jnp.int32),
        pltpu.VMEM((MCH * 3,), jnp.int32),
    ],
)(_body)


def kernel(x, pos, pos_embed):
    x1 = x.reshape(B * S1 * D)
    pos1 = pos.reshape(B * S * 3).astype(jnp.int32)
    pe1 = pos_embed.reshape(NPOS * HALF)
    out1 = _sc_call(x1, pos1, pe1)
    return out1.reshape(B, S1, D)


# speedup vs baseline: 4.1657x; 3.3154x over previous
import functools

import jax
import jax.numpy as jnp
from jax import lax
from jax.experimental import pallas as pl
from jax.experimental.pallas import tpu as pltpu
from jax.experimental.pallas import tpu_sc as plsc

B = 4
S = 8192
S1 = S + 1
D = 1024
HALF = D // 2
NPOS = 101
NC, NS = 2, 16
NW = NC * NS
RPT = S // NW
CH = 16
NCHUNK = RPT // CH
BASER_LEN = 1152
PSTG = 264
MCH = 2048


def _body(x_hbm, p0_hbm, p1_hbm, p2_hbm, pe_hbm, out_hbm,
          petab, baser, xbuf, pstage, pmax):
    wid = lax.axis_index("s") * NC + lax.axis_index("c")
    iota = lax.iota(jnp.int32, 16)

    pltpu.sync_copy(pe_hbm, petab)

    for b in range(B):
        def scan_chunk(c, mv):
            off = pl.multiple_of(b * S + c * MCH, MCH)
            pltpu.sync_copy(p0_hbm.at[pl.ds(off, MCH)], pmax)

            def vmax(u, mv2):
                return jnp.maximum(mv2, pmax[pl.ds(pl.multiple_of(u * 16, 16),
                                                   16)])

            return lax.fori_loop(0, MCH // 16, vmax, mv)

        mvec = lax.fori_loop(0, S // MCH, scan_chunk,
                             jnp.full((16,), -1, jnp.int32))
        m = jnp.max(mvec)
        step = m + 1

        def fill(u, _):
            t = iota + u * 16
            d = (D - 1) - t
            cond = (lax.rem(d, step) == 0) & (d >= 0) & (m > 0)
            baser[pl.ds(pl.multiple_of(b * BASER_LEN + u * 16, 16),
                        16)] = jnp.where(
                cond, 1.0, 0.0).astype(jnp.float32)
            return 0

        lax.fori_loop(0, BASER_LEN // 16, fill, 0)

        pbase = pl.multiple_of(jnp.maximum(wid * RPT - 8, 0), 8)
        boff = wid * RPT - 1 - pbase
        pltpu.sync_copy(p0_hbm.at[pl.ds(pl.multiple_of(b * S + pbase, 8),
                                        PSTG)],
                        pstage.at[pl.ds(0, PSTG)])
        pltpu.sync_copy(p1_hbm.at[pl.ds(pl.multiple_of(b * S + pbase, 8),
                                        PSTG)],
                        pstage.at[pl.ds(PSTG, PSTG)])
        pltpu.sync_copy(p2_hbm.at[pl.ds(pl.multiple_of(b * S + pbase, 8),
                                        PSTG)],
                        pstage.at[pl.ds(2 * PSTG, PSTG)])

        bsplat = jnp.full((16,), b, jnp.int32)

        def do_chunk(k, _):
            r0 = pl.multiple_of(wid * RPT + k * CH, CH)
            pltpu.sync_copy(x_hbm.at[b, pl.ds(r0, CH), :], xbuf)

            start_i = jnp.where((wid == 0) & (k == 0), 1, 0)

            def row(i, _):
                pidx = jnp.full((16,), 0, jnp.int32) + (boff + k * CH + i)
                p0 = plsc.load_gather(pstage, [pidx])
                p1 = plsc.load_gather(pstage, [pidx + PSTG])
                p2 = plsc.load_gather(pstage, [pidx + 2 * PSTG])
                sbase = p0 + b * BASER_LEN
                for v in range(D // 16):
                    ccol = iota + v * 16
                    if v < HALF // 16:
                        pev = plsc.load_gather(petab, [p1 * HALF + ccol])
                    else:
                        pev = plsc.load_gather(petab,
                                               [p2 * HALF + (ccol - HALF)])
                    secv = plsc.load_gather(baser, [sbase + ccol])
                    xv = xbuf[i, pl.ds(v * 16, 16)]
                    xbuf[i, pl.ds(v * 16, 16)] = xv + pev + secv
                return 0

            lax.fori_loop(start_i, CH, row, 0)
            pltpu.sync_copy(xbuf, out_hbm.at[b, pl.ds(r0, CH), :])
            return 0

        lax.fori_loop(0, NCHUNK, do_chunk, 0)

    for b in range(B):
        @pl.when(wid == NW - 4 + b)
        def _last_row():
            pltpu.sync_copy(p0_hbm.at[pl.ds(b * S + S - 8, 8)],
                            pstage.at[pl.ds(0, 8)])
            pltpu.sync_copy(p1_hbm.at[pl.ds(b * S + S - 8, 8)],
                            pstage.at[pl.ds(8, 8)])
            pltpu.sync_copy(p2_hbm.at[pl.ds(b * S + S - 8, 8)],
                            pstage.at[pl.ds(16, 8)])
            pltpu.sync_copy(x_hbm.at[b, pl.ds(S, 1), :],
                            xbuf.at[pl.ds(0, 1), :])
            pidx = jnp.full((16,), 7, jnp.int32)
            p0 = plsc.load_gather(pstage, [pidx])
            p1 = plsc.load_gather(pstage, [pidx + 8])
            p2 = plsc.load_gather(pstage, [pidx + 16])
            sbase = p0 + b * BASER_LEN
            for v in range(D // 16):
                ccol = iota + v * 16
                if v < HALF // 16:
                    pev = plsc.load_gather(petab, [p1 * HALF + ccol])
                else:
                    pev = plsc.load_gather(petab, [p2 * HALF + (ccol - HALF)])
                secv = plsc.load_gather(baser, [sbase + ccol])
                xv = xbuf[0, pl.ds(v * 16, 16)]
                xbuf[0, pl.ds(v * 16, 16)] = xv + pev + secv
            pltpu.sync_copy(xbuf.at[pl.ds(0, 1), :],
                            out_hbm.at[b, pl.ds(S, 1), :])


_sc_call = functools.partial(
    pl.kernel,
    out_type=jax.ShapeDtypeStruct((B, S1, D), jnp.float32),
    mesh=plsc.VectorSubcoreMesh(core_axis_name="c", subcore_axis_name="s"),
    compiler_params=pltpu.CompilerParams(use_tc_tiling_on_sc=True,
                                         needs_layout_passes=False),
    scratch_types=[
        pltpu.VMEM((NPOS * HALF,), jnp.float32),
        pltpu.VMEM((B * BASER_LEN,), jnp.float32),
        pltpu.VMEM((CH, D), jnp.float32),
        pltpu.VMEM((3 * PSTG,), jnp.int32),
        pltpu.VMEM((MCH,), jnp.int32),
    ],
)(_body)


def kernel(x, pos, pos_embed):
    posr = pos.astype(jnp.int32)
    p0 = posr[:, :, 0].reshape(B * S)
    p1 = posr[:, :, 1].reshape(B * S)
    p2 = posr[:, :, 2].reshape(B * S)
    pe1 = pos_embed.reshape(NPOS * HALF)
    return _sc_call(x, p0, p1, p2, pe1)
